# baseline (speedup 1.0000x reference)
import functools

import jax
import jax.numpy as jnp
from jax import lax
from jax.experimental import pallas as pl
from jax.experimental.pallas import tpu as pltpu
from jax.experimental.pallas import tpu_sc as plsc

BATCH = 16384
EMB = 32
HID = 64
_ROWS_PER_SLAB = 4
_SLAB = _ROWS_PER_SLAB * EMB

_NC = 2
_NS = 16
_NW = _NC * _NS
_BPW = BATCH // _NW
_CHUNK = 128
_NCHUNK = _BPW // _CHUNK
_L = 16
_NGRP = _BPW // _L


def _gather_one_table(idx_h, tab_h, out_h, wid, idx_v, slabidx_v, slab_v,
                      rows_v, sem):
    base = wid * _BPW
    pltpu.sync_copy(idx_h.at[pl.ds(base, _BPW)], idx_v)
    for j in range(_NCHUNK):
        for l in range(_CHUNK // _L):
            v = idx_v[pl.ds(j * _CHUNK + l * _L, _L)]
            slabidx_v[j, pl.ds(l * _L, _L)] = lax.shift_right_logical(v, 2)
    copies = [
        pltpu.async_copy(tab_h.at[slabidx_v.at[j]],
                         slab_v.at[pl.ds(j * _CHUNK, _CHUNK)], sem)
        for j in range(_NCHUNK)
    ]
    for c in copies:
        c.wait()

    def grp_body(g, _):
        i_vec = lax.iota(jnp.int32, _L) + g * _L
        iv = idx_v[pl.ds(g * _L, _L)]
        col0 = jnp.bitwise_and(iv, 3) * EMB
        for k in range(EMB):
            vals = plsc.load_gather(slab_v, [i_vec, col0 + k])
            plsc.store_scatter(rows_v, [i_vec, jnp.full((_L,), k, jnp.int32)],
                               vals)
        return 0

    lax.fori_loop(0, _NGRP, grp_body, 0)
    pltpu.sync_copy(rows_v, out_h.at[pl.ds(base, _BPW)])


def _gather_body(sidx_h, ridx_h, stab_h, rtab_h, sout_h, rout_h,
                 idx_v, slabidx_v, slab_v, rows_v, sem):
    wid = lax.axis_index("s") * _NC + lax.axis_index("c")
    _gather_one_table(sidx_h, stab_h, sout_h, wid, idx_v, slabidx_v,
                      slab_v, rows_v, sem)
    _gather_one_table(ridx_h, rtab_h, rout_h, wid, idx_v, slabidx_v,
                      slab_v, rows_v, sem)


_gather_call = functools.partial(
    pl.kernel,
    out_type=[jax.ShapeDtypeStruct((BATCH, EMB), jnp.float32),
              jax.ShapeDtypeStruct((BATCH, EMB), jnp.float32)],
    mesh=plsc.VectorSubcoreMesh(core_axis_name="c", subcore_axis_name="s"),
    scratch_types=[pltpu.VMEM((_BPW,), jnp.int32),
                   pltpu.VMEM((_NCHUNK, _CHUNK), jnp.int32),
                   pltpu.VMEM((_BPW, _SLAB), jnp.float32),
                   pltpu.VMEM((_BPW, EMB), jnp.float32),
                   pltpu.SemaphoreType.DMA],
    compiler_params=pltpu.CompilerParams(use_tc_tiling_on_sc=False,
                                         needs_layout_passes=False),
)(_gather_body)


_BLK = 1024


def _mlp_body(s_ref, r_ref, w1s_ref, w1r_ref, b1_ref, w2_ref, b2_ref, out_ref):
    h = jnp.tanh(
        jnp.dot(s_ref[...], w1s_ref[...], preferred_element_type=jnp.float32)
        + jnp.dot(r_ref[...], w1r_ref[...], preferred_element_type=jnp.float32)
        + b1_ref[...])
    logit = jnp.sum(h * w2_ref[...], axis=1) + b2_ref[0, 0]
    out_ref[...] = jax.nn.sigmoid(logit)


def _mlp_call(s_emb, r_emb, w1s, w1r, b1, w2, b2):
    grid = BATCH // _BLK
    return pl.pallas_call(
        _mlp_body,
        grid=(grid,),
        in_specs=[
            pl.BlockSpec((_BLK, EMB), lambda i: (i, 0)),
            pl.BlockSpec((_BLK, EMB), lambda i: (i, 0)),
            pl.BlockSpec((EMB, HID), lambda i: (0, 0)),
            pl.BlockSpec((EMB, HID), lambda i: (0, 0)),
            pl.BlockSpec((1, HID), lambda i: (0, 0)),
            pl.BlockSpec((1, HID), lambda i: (0, 0)),
            pl.BlockSpec((1, 1), lambda i: (0, 0)),
        ],
        out_specs=pl.BlockSpec((_BLK,), lambda i: (i,)),
        out_shape=jax.ShapeDtypeStruct((BATCH,), jnp.float32),
    )(s_emb, r_emb, w1s, w1r, b1, w2, b2)


def kernel(sender_idx_batch, receiver_idx_batch, sender_table, receiver_table,
           W1, b1, W2, b2):
    sidx = sender_idx_batch.astype(jnp.int32)
    ridx = receiver_idx_batch.astype(jnp.int32)
    stab = sender_table.reshape(-1, _SLAB)
    rtab = receiver_table.reshape(-1, _SLAB)
    s_emb, r_emb = _gather_call(sidx, ridx, stab, rtab)
    w1s = W1[:, :EMB].T
    w1r = W1[:, EMB:].T
    b1r = b1.reshape(1, HID)
    w2r = W2.reshape(1, HID)
    b2r = b2.reshape(1, 1)
    return _mlp_call(s_emb, r_emb, w1s, w1r, b1r, w2r, b2r)


# speedup vs baseline: 1.0143x; 1.0143x over previous
import functools

import jax
import jax.numpy as jnp
from jax import lax
from jax.experimental import pallas as pl
from jax.experimental.pallas import tpu as pltpu
from jax.experimental.pallas import tpu_sc as plsc

BATCH = 16384
EMB = 32
HID = 64
_RPS = 4
_SLAB = _RPS * EMB

_NC = 2
_NS = 16
_NW = _NC * _NS
_BPW = BATCH // _NW
_CHUNK = 128
_NCHUNK = _BPW // _CHUNK
_L = 16
_NGRP = _BPW // _L
_PPW = _BPW // _RPS


def _gather_one_table(idx_h, tab_h, out_h, wid, idx_v, slabidx_v, slab_v,
                      pack_v, sem):
    base = wid * _BPW
    pltpu.sync_copy(idx_h.at[pl.ds(base, _BPW)], idx_v)
    for j in range(_NCHUNK):
        for l in range(_CHUNK // _L):
            v = idx_v[pl.ds(j * _CHUNK + l * _L, _L)]
            slabidx_v[j, pl.ds(l * _L, _L)] = lax.shift_right_logical(v, 2)
    copies = [
        pltpu.async_copy(tab_h.at[slabidx_v.at[j]],
                         slab_v.at[pl.ds(j * _CHUNK, _CHUNK)], sem)
        for j in range(_NCHUNK)
    ]
    for c in copies:
        c.wait()

    def grp_body(g, _):
        i_vec = lax.iota(jnp.int32, _L) + g * _L
        iv = idx_v[pl.ds(g * _L, _L)]
        src_col0 = jnp.bitwise_and(iv, 3) * EMB
        dst_row = lax.shift_right_logical(i_vec, 2)
        dst_col0 = jnp.bitwise_and(i_vec, 3) * EMB
        for k in range(EMB):
            vals = plsc.load_gather(slab_v, [i_vec, src_col0 + k])
            plsc.store_scatter(pack_v, [dst_row, dst_col0 + k], vals)
        return 0

    lax.fori_loop(0, _NGRP, grp_body, 0)
    pltpu.sync_copy(pack_v, out_h.at[pl.ds(wid * _PPW, _PPW)])


def _gather_body(sidx_h, ridx_h, stab_h, rtab_h, sout_h, rout_h,
                 idx_v, slabidx_v, slab_v, pack_v, sem):
    wid = lax.axis_index("s") * _NC + lax.axis_index("c")
    _gather_one_table(sidx_h, stab_h, sout_h, wid, idx_v, slabidx_v,
                      slab_v, pack_v, sem)
    _gather_one_table(ridx_h, rtab_h, rout_h, wid, idx_v, slabidx_v,
                      slab_v, pack_v, sem)


_gather_call = functools.partial(
    pl.kernel,
    out_type=[jax.ShapeDtypeStruct((BATCH // _RPS, _SLAB), jnp.float32),
              jax.ShapeDtypeStruct((BATCH // _RPS, _SLAB), jnp.float32)],
    mesh=plsc.VectorSubcoreMesh(core_axis_name="c", subcore_axis_name="s"),
    scratch_types=[pltpu.VMEM((_BPW,), jnp.int32),
                   pltpu.VMEM((_NCHUNK, _CHUNK), jnp.int32),
                   pltpu.VMEM((_BPW, _SLAB), jnp.float32),
                   pltpu.VMEM((_PPW, _SLAB), jnp.float32),
                   pltpu.SemaphoreType.DMA],
    compiler_params=pltpu.CompilerParams(needs_layout_passes=False),
)(_gather_body)


_BLK = 256
_H4 = _RPS * HID


def _mlp_body(s_ref, r_ref, w1s_ref, w1r_ref, b1_ref, m_ref, b2_ref, out_ref):
    h = jnp.tanh(
        jnp.dot(s_ref[...], w1s_ref[...], preferred_element_type=jnp.float32)
        + jnp.dot(r_ref[...], w1r_ref[...], preferred_element_type=jnp.float32)
        + b1_ref[...])
    logit = jnp.dot(h, m_ref[...], preferred_element_type=jnp.float32)
    out_ref[...] = jax.nn.sigmoid(logit + b2_ref[0, 0])


def _mlp_call(s4, r4, w1s_blk, w1r_blk, b1t, m, b2):
    grid = (BATCH // _RPS) // _BLK
    return pl.pallas_call(
        _mlp_body,
        grid=(grid,),
        in_specs=[
            pl.BlockSpec((_BLK, _SLAB), lambda i: (i, 0)),
            pl.BlockSpec((_BLK, _SLAB), lambda i: (i, 0)),
            pl.BlockSpec((_SLAB, _H4), lambda i: (0, 0)),
            pl.BlockSpec((_SLAB, _H4), lambda i: (0, 0)),
            pl.BlockSpec((1, _H4), lambda i: (0, 0)),
            pl.BlockSpec((_H4, _RPS), lambda i: (0, 0)),
            pl.BlockSpec((1, 1), lambda i: (0, 0)),
        ],
        out_specs=pl.BlockSpec((_BLK, _RPS), lambda i: (i, 0)),
        out_shape=jax.ShapeDtypeStruct((BATCH // _RPS, _RPS), jnp.float32),
    )(s4, r4, w1s_blk, w1r_blk, b1t, m, b2)


def kernel(sender_idx_batch, receiver_idx_batch, sender_table, receiver_table,
           W1, b1, W2, b2):
    sidx = sender_idx_batch.astype(jnp.int32)
    ridx = receiver_idx_batch.astype(jnp.int32)
    stab = sender_table.reshape(-1, _SLAB)
    rtab = receiver_table.reshape(-1, _SLAB)
    s4, r4 = _gather_call(sidx, ridx, stab, rtab)
    eye4 = jnp.eye(_RPS, dtype=jnp.float32)
    w1s_blk = jnp.kron(eye4, W1[:, :EMB].T)
    w1r_blk = jnp.kron(eye4, W1[:, EMB:].T)
    b1t = jnp.tile(b1, _RPS).reshape(1, _H4)
    m = jnp.kron(eye4, W2.T)
    b2r = b2.reshape(1, 1)
    out4 = _mlp_call(s4, r4, w1s_blk, w1r_blk, b1t, m, b2r)
    return out4.reshape(BATCH)


# speedup vs baseline: 1.5389x; 1.5171x over previous
import functools

import jax
import jax.numpy as jnp
from jax import lax
from jax.experimental import pallas as pl
from jax.experimental.pallas import tpu as pltpu
from jax.experimental.pallas import tpu_sc as plsc

BATCH = 16384
EMB = 32
HID = 64

_NC = 2
_NS = 16
_NW = _NC * _NS
_BPW = BATCH // _NW
_LAG = 32


def _gather_one_table(idx_h, tab_h, out_h, wid, idx_v, rows_v, sem):
    base = wid * _BPW
    pltpu.sync_copy(idx_h.at[pl.ds(base, _BPW)], idx_v)

    def wait_one():
        pltpu.make_async_copy(
            tab_h.at[pl.ds(0, 1)], rows_v.at[pl.ds(0, 1)], sem).wait()

    def body(g, _):
        vec = idx_v[pl.ds(g * 16, 16)]
        for l in range(16):
            pltpu.async_copy(tab_h.at[pl.ds(vec[l], 1)],
                             rows_v.at[pl.ds(g * 16 + l, 1)], sem)

        @pl.when(g >= _LAG // 16)
        def _():
            for _i in range(16):
                wait_one()
        return 0

    lax.fori_loop(0, _BPW // 16, body, 0)
    for _ in range(_LAG):
        wait_one()
    pltpu.sync_copy(rows_v, out_h.at[pl.ds(base, _BPW)])


def _gather_body(sidx_h, ridx_h, stab_h, rtab_h, sout_h, rout_h,
                 idx_v, rows_v, sem):
    wid = lax.axis_index("s") * _NC + lax.axis_index("c")
    _gather_one_table(sidx_h, stab_h, sout_h, wid, idx_v, rows_v, sem)
    _gather_one_table(ridx_h, rtab_h, rout_h, wid, idx_v, rows_v, sem)


_gather_call = functools.partial(
    pl.kernel,
    out_type=[jax.ShapeDtypeStruct((BATCH, EMB), jnp.float32),
              jax.ShapeDtypeStruct((BATCH, EMB), jnp.float32)],
    mesh=plsc.VectorSubcoreMesh(core_axis_name="c", subcore_axis_name="s"),
    scratch_types=[pltpu.VMEM((_BPW,), jnp.int32),
                   pltpu.VMEM((_BPW, EMB), jnp.float32),
                   pltpu.SemaphoreType.DMA],
    compiler_params=pltpu.CompilerParams(needs_layout_passes=False),
)(_gather_body)


_BLK = 1024


def _mlp_body(s_ref, r_ref, w1s_ref, w1r_ref, b1_ref, w2_ref, b2_ref, out_ref):
    h = jnp.tanh(
        jnp.dot(s_ref[...], w1s_ref[...], preferred_element_type=jnp.float32)
        + jnp.dot(r_ref[...], w1r_ref[...], preferred_element_type=jnp.float32)
        + b1_ref[...])
    logit = jnp.sum(h * w2_ref[...], axis=1) + b2_ref[0, 0]
    out_ref[...] = jax.nn.sigmoid(logit)


def _mlp_call(s_emb, r_emb, w1s, w1r, b1, w2, b2):
    grid = BATCH // _BLK
    return pl.pallas_call(
        _mlp_body,
        grid=(grid,),
        in_specs=[
            pl.BlockSpec((_BLK, EMB), lambda i: (i, 0)),
            pl.BlockSpec((_BLK, EMB), lambda i: (i, 0)),
            pl.BlockSpec((EMB, HID), lambda i: (0, 0)),
            pl.BlockSpec((EMB, HID), lambda i: (0, 0)),
            pl.BlockSpec((1, HID), lambda i: (0, 0)),
            pl.BlockSpec((1, HID), lambda i: (0, 0)),
            pl.BlockSpec((1, 1), lambda i: (0, 0)),
        ],
        out_specs=pl.BlockSpec((_BLK,), lambda i: (i,)),
        out_shape=jax.ShapeDtypeStruct((BATCH,), jnp.float32),
    )(s_emb, r_emb, w1s, w1r, b1, w2, b2)


def kernel(sender_idx_batch, receiver_idx_batch, sender_table, receiver_table,
           W1, b1, W2, b2):
    sidx = sender_idx_batch.astype(jnp.int32)
    ridx = receiver_idx_batch.astype(jnp.int32)
    s_emb, r_emb = _gather_call(sidx, ridx, sender_table, receiver_table)
    w1s = W1[:, :EMB].T
    w1r = W1[:, EMB:].T
    b1r = b1.reshape(1, HID)
    w2r = W2.reshape(1, HID)
    b2r = b2.reshape(1, 1)
    return _mlp_call(s_emb, r_emb, w1s, w1r, b1r, w2r, b2r)
